# Initial kernel scaffold; baseline (speedup 1.0000x reference)
import jax
import jax.numpy as jnp
from jax.experimental import pallas as pl


def kernel(x, edge_index, Wl0, Wr0, b0, Wl1, Wr1, b1, lin_w, lin_b):
    raise NotImplementedError("write your pallas kernel here")


# speedup vs baseline: 5.5648x; 5.5648x over previous
import functools

import jax
import jax.numpy as jnp
from jax import lax
from jax.experimental import pallas as pl
from jax.experimental.pallas import tpu as pltpu
from jax.experimental.pallas import tpu_sc as plsc

N = 10000
E = 320000
D = 128
H = 64

NC = 2
NS = 16
NW = NC * NS
CH = 128
G = 80
E_PAD = NW * G * CH
N_PAD = 10240
ROWS_PER_TILE = N_PAD // NS

_F32 = jnp.float32
_HIGH = jax.lax.Precision.HIGHEST


def _dot(a, b):
    return jnp.dot(a, b, preferred_element_type=_F32, precision=_HIGH)


def _sc_edge_pass(with_deg: bool):
    out_type = [jax.ShapeDtypeStruct((NC, N_PAD, H), _F32)]
    if with_deg:
        out_type.append(jax.ShapeDtypeStruct((NC, N_PAD, 16), _F32))

    scratch = [
        pltpu.VMEM_SHARED((N_PAD, H), _F32),
        pltpu.VMEM((G, CH), jnp.int32),
        pltpu.VMEM((G, CH), jnp.int32),
        pltpu.VMEM((CH, H), _F32),
        pltpu.SemaphoreType.DMA,
    ]
    if with_deg:
        scratch.insert(1, pltpu.VMEM_SHARED((N_PAD, 16), _F32))
        scratch.append(pltpu.VMEM((CH, 16), _F32))

    mesh = plsc.VectorSubcoreMesh(core_axis_name="c", subcore_axis_name="s")

    @functools.partial(
        pl.kernel, out_type=tuple(out_type), mesh=mesh,
        scratch_types=tuple(scratch),
        compiler_params=pltpu.CompilerParams(use_tc_tiling_on_sc=False))
    def body(*refs):
        if with_deg:
            (y_hbm, srcp, dstp, z_hbm, z16_hbm, ones_hbm, acc_out, deg_out,
             acc_sh, deg_sh, src_v, dst_v, rows_v, sem, ones_v) = refs
        else:
            (y_hbm, srcp, dstp, z_hbm, z16_hbm, ones_hbm, acc_out,
             acc_sh, src_v, dst_v, rows_v, sem) = refs

        c = lax.axis_index("c")
        s = lax.axis_index("s")
        w = c * NS + s
        row0 = s * ROWS_PER_TILE

        pltpu.sync_copy(z_hbm.at[pl.ds(row0, ROWS_PER_TILE)],
                        acc_sh.at[pl.ds(row0, ROWS_PER_TILE)])
        if with_deg:
            pltpu.sync_copy(z16_hbm.at[pl.ds(row0, ROWS_PER_TILE)],
                            deg_sh.at[pl.ds(row0, ROWS_PER_TILE)])
            pltpu.sync_copy(ones_hbm, ones_v)

        pltpu.sync_copy(srcp.at[w], src_v)
        pltpu.sync_copy(dstp.at[w], dst_v)
        plsc.subcore_barrier()

        def chunk(g, _):
            pltpu.async_copy(y_hbm.at[src_v.at[g]], rows_v, sem).wait()
            pltpu.sync_copy(rows_v, acc_sh.at[dst_v.at[g]], add=True)
            if with_deg:
                pltpu.sync_copy(ones_v, deg_sh.at[dst_v.at[g]], add=True)
            return 0

        lax.fori_loop(0, G, chunk, 0)
        plsc.subcore_barrier()

        pltpu.sync_copy(acc_sh.at[pl.ds(row0, ROWS_PER_TILE)],
                        acc_out.at[c, pl.ds(row0, ROWS_PER_TILE)])
        if with_deg:
            pltpu.sync_copy(deg_sh.at[pl.ds(row0, ROWS_PER_TILE)],
                            deg_out.at[c, pl.ds(row0, ROWS_PER_TILE)])

    return body


_sc_pass0 = _sc_edge_pass(with_deg=True)
_sc_pass1 = _sc_edge_pass(with_deg=False)


def _pre_body(x_ref, wl_ref, wr_ref, b_ref, y_ref, r_ref):
    x = x_ref[...]
    y_ref[...] = _dot(x, wl_ref[...])
    r_ref[...] = _dot(x, wr_ref[...]) + b_ref[...]


def _mid_body(acc_ref, deg_ref, r_ref, wl_ref, wr_ref, b_ref, y_ref, r1_ref):
    acc = acc_ref[0, :N, :] + acc_ref[1, :N, :]
    deg = deg_ref[0, :N, 0:1] + deg_ref[1, :N, 0:1]
    mean = acc / jnp.maximum(deg, 1.0)
    h = jnp.maximum(mean + r_ref[...], 0.0)
    y_ref[...] = _dot(h, wl_ref[...])
    r1_ref[...] = _dot(h, wr_ref[...]) + b_ref[...]


def _fin_body(acc_ref, deg_ref, r_ref, w_ref, b_ref, o_ref):
    acc = acc_ref[0, :N, :] + acc_ref[1, :N, :]
    deg = deg_ref[0, :N, 0:1] + deg_ref[1, :N, 0:1]
    mean = acc / jnp.maximum(deg, 1.0)
    h = jnp.maximum(mean + r_ref[...], 0.0)
    o_ref[...] = _dot(h, w_ref[...]) + b_ref[...]


_pre_call = pl.pallas_call(
    _pre_body,
    out_shape=(jax.ShapeDtypeStruct((N, H), _F32),
               jax.ShapeDtypeStruct((N, H), _F32)))

_mid_call = pl.pallas_call(
    _mid_body,
    out_shape=(jax.ShapeDtypeStruct((N, H), _F32),
               jax.ShapeDtypeStruct((N, H), _F32)))

_fin_call = pl.pallas_call(
    _fin_body,
    out_shape=jax.ShapeDtypeStruct((N, 1), _F32))


def kernel(x, edge_index, Wl0, Wr0, b0, Wl1, Wr1, b1, lin_w, lin_b):
    src = edge_index[0]
    dst = edge_index[1]
    srcp = jnp.concatenate(
        [src, jnp.zeros((E_PAD - E,), jnp.int32)]).reshape(NW, G, CH)
    dstp = jnp.concatenate(
        [dst, jnp.full((E_PAD - E,), N, jnp.int32)]).reshape(NW, G, CH)

    zeros = jnp.zeros((N_PAD, H), _F32)
    zeros16 = jnp.zeros((N_PAD, 16), _F32)
    ones = jnp.ones((CH, 16), _F32)

    y0, r0 = _pre_call(x, Wl0, Wr0, b0.reshape(1, H))
    acc0, deg0 = _sc_pass0(y0, srcp, dstp, zeros, zeros16, ones)
    y1, r1 = _mid_call(acc0, deg0, r0, Wl1, Wr1, b1.reshape(1, H))
    (acc1,) = _sc_pass1(y1, srcp, dstp, zeros, zeros16, ones)
    out = _fin_call(acc1, deg0, r1, lin_w, lin_b.reshape(1, 1))
    return out[:, 0]


# speedup vs baseline: 5.7532x; 1.0339x over previous
import functools

import jax
import jax.numpy as jnp
from jax import lax
from jax.experimental import pallas as pl
from jax.experimental.pallas import tpu as pltpu
from jax.experimental.pallas import tpu_sc as plsc

N = 10000
E = 320000
D = 128
H = 64

NC = 2
NS = 16
NW = NC * NS
CH = 128
G = 80
E_PAD = NW * G * CH
N_PAD = 10240
ROWS_PER_TILE = N_PAD // NS
NB = 8
K = 4

_F32 = jnp.float32
_HIGH = jax.lax.Precision.HIGHEST


def _dot(a, b):
    return jnp.dot(a, b, preferred_element_type=_F32, precision=_HIGH)


_MESH = plsc.VectorSubcoreMesh(core_axis_name="c", subcore_axis_name="s")
_SC_PARAMS = pltpu.CompilerParams(use_tc_tiling_on_sc=False)


@functools.partial(
    pl.kernel,
    out_type=jax.ShapeDtypeStruct((NC, N_PAD, H), _F32),
    mesh=_MESH,
    scratch_types=(
        pltpu.VMEM_SHARED((N_PAD, H), _F32),
        pltpu.VMEM((G, CH), jnp.int32),
        pltpu.VMEM((G, CH), jnp.int32),
        pltpu.VMEM((NB, CH, H), _F32),
        pltpu.SemaphoreType.DMA((NB,)),
        pltpu.SemaphoreType.DMA((NB,)),
    ),
    compiler_params=_SC_PARAMS)
def _sc_edge_pass(y_hbm, srcp, dstp, z_hbm, acc_out,
                  acc_sh, src_v, dst_v, rows_v, gsem, ssem):
    c = lax.axis_index("c")
    s = lax.axis_index("s")
    w = c * NS + s
    row0 = s * ROWS_PER_TILE

    pltpu.sync_copy(z_hbm.at[pl.ds(row0, ROWS_PER_TILE)],
                    acc_sh.at[pl.ds(row0, ROWS_PER_TILE)])
    pltpu.sync_copy(srcp.at[w], src_v)
    pltpu.sync_copy(dstp.at[w], dst_v)
    plsc.subcore_barrier()

    for b in range(K):
        pltpu.async_copy(y_hbm.at[src_v.at[b]], rows_v.at[b], gsem.at[b])

    def chunk(g, _):
        g2 = g + K
        s2 = lax.rem(g2, NB)

        @pl.when(g2 < G)
        def _prefetch():
            @pl.when(g >= K)
            def _drain_slot():
                pltpu.make_async_copy(
                    rows_v.at[s2], acc_sh.at[dst_v.at[g - K]],
                    ssem.at[s2]).wait()
            pltpu.async_copy(y_hbm.at[src_v.at[g2]], rows_v.at[s2],
                             gsem.at[s2])

        b = lax.rem(g, NB)
        pltpu.make_async_copy(y_hbm.at[src_v.at[g]], rows_v.at[b],
                              gsem.at[b]).wait()
        pltpu.async_copy(rows_v.at[b], acc_sh.at[dst_v.at[g]],
                         ssem.at[b], add=True)
        return 0

    lax.fori_loop(0, G, chunk, 0)
    for i in range(NB):
        pltpu.make_async_copy(rows_v.at[i], acc_sh.at[dst_v.at[G - NB + i]],
                              ssem.at[i]).wait()
    plsc.subcore_barrier()

    pltpu.sync_copy(acc_sh.at[pl.ds(row0, ROWS_PER_TILE)],
                    acc_out.at[c, pl.ds(row0, ROWS_PER_TILE)])


@functools.partial(
    pl.kernel,
    out_type=jax.ShapeDtypeStruct((NC, N_PAD, 16), _F32),
    mesh=_MESH,
    scratch_types=(
        pltpu.VMEM_SHARED((N_PAD, 16), _F32),
        pltpu.VMEM((G, CH), jnp.int32),
        pltpu.VMEM((CH, 16), _F32),
        pltpu.SemaphoreType.DMA,
    ),
    compiler_params=_SC_PARAMS)
def _sc_deg_pass(dstp, z16_hbm, ones_hbm, deg_out, deg_sh, dst_v, ones_v,
                 dsem):
    c = lax.axis_index("c")
    s = lax.axis_index("s")
    w = c * NS + s
    row0 = s * ROWS_PER_TILE

    pltpu.sync_copy(z16_hbm.at[pl.ds(row0, ROWS_PER_TILE)],
                    deg_sh.at[pl.ds(row0, ROWS_PER_TILE)])
    pltpu.sync_copy(dstp.at[w], dst_v)
    pltpu.sync_copy(ones_hbm, ones_v)
    plsc.subcore_barrier()

    def fire(g, _):
        pltpu.async_copy(ones_v, deg_sh.at[dst_v.at[g]], dsem, add=True)
        return 0

    lax.fori_loop(0, G, fire, 0)

    def drain(g, _):
        pltpu.make_async_copy(ones_v, deg_sh.at[dst_v.at[g]], dsem).wait()
        return 0

    lax.fori_loop(0, G, drain, 0)
    plsc.subcore_barrier()

    pltpu.sync_copy(deg_sh.at[pl.ds(row0, ROWS_PER_TILE)],
                    deg_out.at[c, pl.ds(row0, ROWS_PER_TILE)])


def _pre_body(x_ref, wl_ref, wr_ref, b_ref, y_ref, r_ref):
    x = x_ref[...]
    y_ref[...] = _dot(x, wl_ref[...])
    r_ref[...] = _dot(x, wr_ref[...]) + b_ref[...]


def _mid_body(acc_ref, deg_ref, r_ref, wl_ref, wr_ref, b_ref, y_ref, r1_ref):
    acc = acc_ref[0, :N, :] + acc_ref[1, :N, :]
    deg = deg_ref[0, :N, 0:1] + deg_ref[1, :N, 0:1]
    mean = acc / jnp.maximum(deg, 1.0)
    h = jnp.maximum(mean + r_ref[...], 0.0)
    y_ref[...] = _dot(h, wl_ref[...])
    r1_ref[...] = _dot(h, wr_ref[...]) + b_ref[...]


def _fin_body(acc_ref, deg_ref, r_ref, w_ref, b_ref, o_ref):
    acc = acc_ref[0, :N, :] + acc_ref[1, :N, :]
    deg = deg_ref[0, :N, 0:1] + deg_ref[1, :N, 0:1]
    mean = acc / jnp.maximum(deg, 1.0)
    h = jnp.maximum(mean + r_ref[...], 0.0)
    o_ref[...] = _dot(h, w_ref[...]) + b_ref[...]


_pre_call = pl.pallas_call(
    _pre_body,
    out_shape=(jax.ShapeDtypeStruct((N, H), _F32),
               jax.ShapeDtypeStruct((N, H), _F32)))

_mid_call = pl.pallas_call(
    _mid_body,
    out_shape=(jax.ShapeDtypeStruct((N, H), _F32),
               jax.ShapeDtypeStruct((N, H), _F32)))

_fin_call = pl.pallas_call(
    _fin_body,
    out_shape=jax.ShapeDtypeStruct((N, 1), _F32))


def kernel(x, edge_index, Wl0, Wr0, b0, Wl1, Wr1, b1, lin_w, lin_b):
    src = edge_index[0]
    dst = edge_index[1]
    srcp = jnp.concatenate(
        [src, jnp.zeros((E_PAD - E,), jnp.int32)]).reshape(NW, G, CH)
    dstp = jnp.concatenate(
        [dst, jnp.full((E_PAD - E,), N, jnp.int32)]).reshape(NW, G, CH)

    zeros = jnp.zeros((N_PAD, H), _F32)
    zeros16 = jnp.zeros((N_PAD, 16), _F32)
    ones = jnp.ones((CH, 16), _F32)

    deg0 = _sc_deg_pass(dstp, zeros16, ones)
    y0, r0 = _pre_call(x, Wl0, Wr0, b0.reshape(1, H))
    acc0 = _sc_edge_pass(y0, srcp, dstp, zeros)
    y1, r1 = _mid_call(acc0, deg0, r0, Wl1, Wr1, b1.reshape(1, H))
    acc1 = _sc_edge_pass(y1, srcp, dstp, zeros)
    out = _fin_call(acc1, deg0, r1, lin_w, lin_b.reshape(1, 1))
    return out[:, 0]


# speedup vs baseline: 5.8147x; 1.0107x over previous
import functools

import jax
import jax.numpy as jnp
from jax import lax
from jax.experimental import pallas as pl
from jax.experimental.pallas import tpu as pltpu
from jax.experimental.pallas import tpu_sc as plsc

N = 10000
E = 320000
D = 128
H = 64

NC = 2
NS = 16
NW = NC * NS
CH = 128
G = 80
E_PAD = NW * G * CH
N_PAD = 10240
ROWS_PER_TILE = N_PAD // NS
NB = 8
K = 4

_F32 = jnp.float32
_HIGH = jax.lax.Precision.HIGHEST


def _dot(a, b):
    return jnp.dot(a, b, preferred_element_type=_F32, precision=_HIGH)


_MESH = plsc.VectorSubcoreMesh(core_axis_name="c", subcore_axis_name="s")
_SC_PARAMS = pltpu.CompilerParams(use_tc_tiling_on_sc=False)


@functools.partial(
    pl.kernel,
    out_type=jax.ShapeDtypeStruct((NC, N_PAD, H), _F32),
    mesh=_MESH,
    scratch_types=(
        pltpu.VMEM_SHARED((N_PAD, H), _F32),
        pltpu.VMEM((G, CH), jnp.int32),
        pltpu.VMEM((G, CH), jnp.int32),
        pltpu.VMEM((NB, CH, H), _F32),
        pltpu.SemaphoreType.DMA((NB,)),
        pltpu.SemaphoreType.DMA((NB,)),
    ),
    compiler_params=_SC_PARAMS)
def _sc_edge_pass(y_hbm, srcp, dstp, z_hbm, acc_out,
                  acc_sh, src_v, dst_v, rows_v, gsem, ssem):
    c = lax.axis_index("c")
    s = lax.axis_index("s")
    w = c * NS + s
    row0 = s * ROWS_PER_TILE

    pltpu.sync_copy(z_hbm.at[pl.ds(row0, ROWS_PER_TILE)],
                    acc_sh.at[pl.ds(row0, ROWS_PER_TILE)])
    pltpu.sync_copy(srcp.at[w], src_v)
    pltpu.sync_copy(dstp.at[w], dst_v)
    plsc.subcore_barrier()

    for b in range(K):
        pltpu.async_copy(y_hbm.at[src_v.at[b]], rows_v.at[b], gsem.at[b])

    def chunk(g, _):
        g2 = g + K
        s2 = lax.rem(g2, NB)

        @pl.when(g2 < G)
        def _prefetch():
            @pl.when(g >= K)
            def _drain_slot():
                pltpu.make_async_copy(
                    rows_v.at[s2], acc_sh.at[dst_v.at[g - K]],
                    ssem.at[s2]).wait()
            pltpu.async_copy(y_hbm.at[src_v.at[g2]], rows_v.at[s2],
                             gsem.at[s2])

        b = lax.rem(g, NB)
        pltpu.make_async_copy(y_hbm.at[src_v.at[g]], rows_v.at[b],
                              gsem.at[b]).wait()
        pltpu.async_copy(rows_v.at[b], acc_sh.at[dst_v.at[g]],
                         ssem.at[b], add=True)
        return 0

    lax.fori_loop(0, G, chunk, 0)
    for i in range(NB):
        pltpu.make_async_copy(rows_v.at[i], acc_sh.at[dst_v.at[G - NB + i]],
                              ssem.at[i]).wait()
    plsc.subcore_barrier()

    pltpu.sync_copy(acc_sh.at[pl.ds(row0, ROWS_PER_TILE)],
                    acc_out.at[c, pl.ds(row0, ROWS_PER_TILE)])


@functools.partial(
    pl.kernel,
    out_type=jax.ShapeDtypeStruct((NC, N_PAD, 16), _F32),
    mesh=_MESH,
    scratch_types=(
        pltpu.VMEM_SHARED((N_PAD, 16), _F32),
        pltpu.VMEM((G, CH), jnp.int32),
        pltpu.VMEM((CH, 16), _F32),
        pltpu.SemaphoreType.DMA,
    ),
    compiler_params=_SC_PARAMS)
def _sc_deg_pass(dstp, z16_hbm, ones_hbm, deg_out, deg_sh, dst_v, ones_v,
                 dsem):
    c = lax.axis_index("c")
    s = lax.axis_index("s")
    w = c * NS + s
    row0 = s * ROWS_PER_TILE

    pltpu.sync_copy(z16_hbm.at[pl.ds(row0, ROWS_PER_TILE)],
                    deg_sh.at[pl.ds(row0, ROWS_PER_TILE)])
    pltpu.sync_copy(dstp.at[w], dst_v)
    pltpu.sync_copy(ones_hbm, ones_v)
    plsc.subcore_barrier()

    def fire(g, _):
        pltpu.async_copy(ones_v, deg_sh.at[dst_v.at[g]], dsem, add=True)
        return 0

    lax.fori_loop(0, G, fire, 0)

    def drain(g, _):
        pltpu.make_async_copy(ones_v, deg_sh.at[dst_v.at[g]], dsem).wait()
        return 0

    lax.fori_loop(0, G, drain, 0)
    plsc.subcore_barrier()

    pltpu.sync_copy(deg_sh.at[pl.ds(row0, ROWS_PER_TILE)],
                    deg_out.at[c, pl.ds(row0, ROWS_PER_TILE)])


def _pre_body(x_ref, wl_ref, wr_ref, b_ref, y_ref, r_ref):
    x = x_ref[...]
    y_ref[...] = _dot(x, wl_ref[...])
    r_ref[...] = _dot(x, wr_ref[...]) + b_ref[...]


def _mid_body(acc_ref, deg_ref, r_ref, wl_ref, wr_ref, b_ref, y_ref, r1_ref):
    acc = acc_ref[0, :N, :] + acc_ref[1, :N, :]
    deg = deg_ref[0, :N, 0:1] + deg_ref[1, :N, 0:1]
    mean = acc / jnp.maximum(deg, 1.0)
    h = jnp.maximum(mean + r_ref[...], 0.0)
    y_ref[...] = _dot(h, wl_ref[...])
    r1_ref[...] = _dot(h, wr_ref[...]) + b_ref[...]


def _fin_body(acc_ref, deg_ref, r_ref, w_ref, b_ref, o_ref):
    acc = acc_ref[0, :N, :] + acc_ref[1, :N, :]
    deg = deg_ref[0, :N, 0:1] + deg_ref[1, :N, 0:1]
    mean = acc / jnp.maximum(deg, 1.0)
    h = jnp.maximum(mean + r_ref[...], 0.0)
    o_ref[...] = _dot(h, w_ref[...]) + b_ref[...]


_pre_call = pl.pallas_call(
    _pre_body,
    out_shape=(jax.ShapeDtypeStruct((N, H), _F32),
               jax.ShapeDtypeStruct((N, H), _F32)))

_mid_call = pl.pallas_call(
    _mid_body,
    out_shape=(jax.ShapeDtypeStruct((N, H), _F32),
               jax.ShapeDtypeStruct((N, H), _F32)))

_fin_call = pl.pallas_call(
    _fin_body,
    out_shape=jax.ShapeDtypeStruct((N, 1), _F32))


def kernel(x, edge_index, Wl0, Wr0, b0, Wl1, Wr1, b1, lin_w, lin_b):
    src = edge_index[0]
    dst = edge_index[1]
    pad_dst = N + jnp.arange(E_PAD - E, dtype=jnp.int32) % (N_PAD - N)
    srcp = jnp.concatenate(
        [src, jnp.zeros((E_PAD - E,), jnp.int32)]).reshape(NW, G, CH)
    dstp = jnp.concatenate([dst, pad_dst]).reshape(NW, G, CH)

    zeros = jnp.zeros((N_PAD, H), _F32)
    zeros16 = jnp.zeros((N_PAD, 16), _F32)
    ones = jnp.ones((CH, 16), _F32)

    deg0 = _sc_deg_pass(dstp, zeros16, ones)
    y0, r0 = _pre_call(x, Wl0, Wr0, b0.reshape(1, H))
    acc0 = _sc_edge_pass(y0, srcp, dstp, zeros)
    y1, r1 = _mid_call(acc0, deg0, r0, Wl1, Wr1, b1.reshape(1, H))
    acc1 = _sc_edge_pass(y1, srcp, dstp, zeros)
    out = _fin_call(acc1, deg0, r1, lin_w, lin_b.reshape(1, 1))
    return out[:, 0]


# speedup vs baseline: 13.6402x; 2.3458x over previous
import functools

import jax
import jax.numpy as jnp
from jax import lax
from jax.experimental import pallas as pl
from jax.experimental.pallas import tpu as pltpu
from jax.experimental.pallas import tpu_sc as plsc

N = 10000
E = 320000
D = 128
H = 64
HH = H // 2

NC = 2
NS = 16
NW = NC * NS
CH = 128
GD = 80
G = 160
E_PAD = NS * G * CH
N_PAD = 10240
ROWS_PER_TILE = N_PAD // NS
YROWS_PER_TILE = N // NS
NB = 8
K = 4

_F32 = jnp.float32
_HIGH = jax.lax.Precision.HIGHEST


def _dot(a, b):
    return jnp.dot(a, b, preferred_element_type=_F32, precision=_HIGH)


_MESH = plsc.VectorSubcoreMesh(core_axis_name="c", subcore_axis_name="s")
_SC_PARAMS = pltpu.CompilerParams(use_tc_tiling_on_sc=False)


@functools.partial(
    pl.kernel,
    out_type=jax.ShapeDtypeStruct((NC, N_PAD, HH), _F32),
    mesh=_MESH,
    scratch_types=(
        pltpu.VMEM_SHARED((N_PAD, HH), _F32),
        pltpu.VMEM_SHARED((N, HH), _F32),
        pltpu.VMEM((G, CH), jnp.int32),
        pltpu.VMEM((G, CH), jnp.int32),
        pltpu.VMEM((NB, CH, HH), _F32),
        pltpu.SemaphoreType.DMA((NB,)),
        pltpu.SemaphoreType.DMA((NB,)),
    ),
    compiler_params=_SC_PARAMS)
def _sc_edge_pass(y_hbm, srcp, dstp, z_hbm, acc_out,
                  acc_sh, y_sh, src_v, dst_v, rows_v, gsem, ssem):
    c = lax.axis_index("c")
    s = lax.axis_index("s")
    row0 = s * ROWS_PER_TILE
    yrow0 = s * YROWS_PER_TILE

    pltpu.sync_copy(z_hbm.at[pl.ds(row0, ROWS_PER_TILE)],
                    acc_sh.at[pl.ds(row0, ROWS_PER_TILE)])
    pltpu.sync_copy(y_hbm.at[c, pl.ds(yrow0, YROWS_PER_TILE)],
                    y_sh.at[pl.ds(yrow0, YROWS_PER_TILE)])
    pltpu.sync_copy(srcp.at[s], src_v)
    pltpu.sync_copy(dstp.at[s], dst_v)
    plsc.subcore_barrier()

    for b in range(K):
        pltpu.async_copy(y_sh.at[src_v.at[b]], rows_v.at[b], gsem.at[b])

    def chunk(g, _):
        g2 = g + K
        s2 = lax.rem(g2, NB)

        @pl.when(g2 < G)
        def _prefetch():
            @pl.when(g >= K)
            def _drain_slot():
                pltpu.make_async_copy(
                    rows_v.at[s2], acc_sh.at[dst_v.at[g - K]],
                    ssem.at[s2]).wait()
            pltpu.async_copy(y_sh.at[src_v.at[g2]], rows_v.at[s2],
                             gsem.at[s2])

        b = lax.rem(g, NB)
        pltpu.make_async_copy(y_sh.at[src_v.at[g]], rows_v.at[b],
                              gsem.at[b]).wait()
        pltpu.async_copy(rows_v.at[b], acc_sh.at[dst_v.at[g]],
                         ssem.at[b], add=True)
        return 0

    lax.fori_loop(0, G, chunk, 0)
    for i in range(NB):
        pltpu.make_async_copy(rows_v.at[i], acc_sh.at[dst_v.at[G - NB + i]],
                              ssem.at[i]).wait()
    plsc.subcore_barrier()

    pltpu.sync_copy(acc_sh.at[pl.ds(row0, ROWS_PER_TILE)],
                    acc_out.at[c, pl.ds(row0, ROWS_PER_TILE)])


@functools.partial(
    pl.kernel,
    out_type=jax.ShapeDtypeStruct((NC, N_PAD, 16), _F32),
    mesh=_MESH,
    scratch_types=(
        pltpu.VMEM_SHARED((N_PAD, 16), _F32),
        pltpu.VMEM((GD, CH), jnp.int32),
        pltpu.VMEM((CH, 16), _F32),
        pltpu.SemaphoreType.DMA,
    ),
    compiler_params=_SC_PARAMS)
def _sc_deg_pass(dstp, z16_hbm, ones_hbm, deg_out, deg_sh, dst_v, ones_v,
                 dsem):
    c = lax.axis_index("c")
    s = lax.axis_index("s")
    w = c * NS + s
    row0 = s * ROWS_PER_TILE

    pltpu.sync_copy(z16_hbm.at[pl.ds(row0, ROWS_PER_TILE)],
                    deg_sh.at[pl.ds(row0, ROWS_PER_TILE)])
    pltpu.sync_copy(dstp.at[w], dst_v)
    pltpu.sync_copy(ones_hbm, ones_v)
    plsc.subcore_barrier()

    def fire(g, _):
        pltpu.async_copy(ones_v, deg_sh.at[dst_v.at[g]], dsem, add=True)
        return 0

    lax.fori_loop(0, GD, fire, 0)

    def drain(g, _):
        pltpu.make_async_copy(ones_v, deg_sh.at[dst_v.at[g]], dsem).wait()
        return 0

    lax.fori_loop(0, GD, drain, 0)
    plsc.subcore_barrier()

    pltpu.sync_copy(deg_sh.at[pl.ds(row0, ROWS_PER_TILE)],
                    deg_out.at[c, pl.ds(row0, ROWS_PER_TILE)])


def _pre_body(x_ref, wl_ref, wr_ref, b_ref, y_ref, r_ref):
    x = x_ref[...]
    y = _dot(x, wl_ref[...])
    y_ref[0] = y[:, :HH]
    y_ref[1] = y[:, HH:]
    r_ref[...] = _dot(x, wr_ref[...]) + b_ref[...]


def _mid_body(acc_ref, deg_ref, r_ref, wl_ref, wr_ref, b_ref, y_ref, r1_ref):
    acc = jnp.concatenate([acc_ref[0], acc_ref[1]], axis=1)
    deg = deg_ref[0, :, 0:1] + deg_ref[1, :, 0:1]
    mean = acc / jnp.maximum(deg, 1.0)
    h = jnp.maximum(mean + r_ref[...], 0.0)
    y = _dot(h, wl_ref[...])
    y_ref[0] = y[:, :HH]
    y_ref[1] = y[:, HH:]
    r1_ref[...] = _dot(h, wr_ref[...]) + b_ref[...]


def _fin_body(acc_ref, deg_ref, r_ref, w_ref, b_ref, o_ref):
    acc = jnp.concatenate([acc_ref[0], acc_ref[1]], axis=1)
    deg = deg_ref[0, :, 0:1] + deg_ref[1, :, 0:1]
    mean = acc / jnp.maximum(deg, 1.0)
    h = jnp.maximum(mean + r_ref[...], 0.0)
    o_ref[...] = _dot(h, w_ref[...]) + b_ref[...]


_pre_call = pl.pallas_call(
    _pre_body,
    out_shape=(jax.ShapeDtypeStruct((NC, N, HH), _F32),
               jax.ShapeDtypeStruct((N, H), _F32)))

_B = 2000

_mid_call = pl.pallas_call(
    _mid_body,
    grid=(N // _B,),
    in_specs=[
        pl.BlockSpec((NC, _B, HH), lambda i: (0, i, 0)),
        pl.BlockSpec((NC, _B, 16), lambda i: (0, i, 0)),
        pl.BlockSpec((_B, H), lambda i: (i, 0)),
        pl.BlockSpec((H, H), lambda i: (0, 0)),
        pl.BlockSpec((H, H), lambda i: (0, 0)),
        pl.BlockSpec((1, H), lambda i: (0, 0)),
    ],
    out_specs=(pl.BlockSpec((NC, _B, HH), lambda i: (0, i, 0)),
               pl.BlockSpec((_B, H), lambda i: (i, 0))),
    out_shape=(jax.ShapeDtypeStruct((NC, N, HH), _F32),
               jax.ShapeDtypeStruct((N, H), _F32)))

_fin_call = pl.pallas_call(
    _fin_body,
    grid=(N // _B,),
    in_specs=[
        pl.BlockSpec((NC, _B, HH), lambda i: (0, i, 0)),
        pl.BlockSpec((NC, _B, 16), lambda i: (0, i, 0)),
        pl.BlockSpec((_B, H), lambda i: (i, 0)),
        pl.BlockSpec((H, 1), lambda i: (0, 0)),
        pl.BlockSpec((1, 1), lambda i: (0, 0)),
    ],
    out_specs=pl.BlockSpec((_B, 1), lambda i: (i, 0)),
    out_shape=jax.ShapeDtypeStruct((N, 1), _F32))


def kernel(x, edge_index, Wl0, Wr0, b0, Wl1, Wr1, b1, lin_w, lin_b):
    src = edge_index[0]
    dst = edge_index[1]
    pad_dst = N + jnp.arange(E_PAD - E, dtype=jnp.int32) % (N_PAD - N)
    src_flat = jnp.concatenate([src, jnp.zeros((E_PAD - E,), jnp.int32)])
    dst_flat = jnp.concatenate([dst, pad_dst])
    srcp = src_flat.reshape(NS, G, CH)
    dstp = dst_flat.reshape(NS, G, CH)
    dstp_deg = dst_flat.reshape(NW, GD, CH)

    zeros = jnp.zeros((N_PAD, HH), _F32)
    zeros16 = jnp.zeros((N_PAD, 16), _F32)
    ones = jnp.ones((CH, 16), _F32)

    deg0 = _sc_deg_pass(dstp_deg, zeros16, ones)
    y0, r0 = _pre_call(x, Wl0, Wr0, b0.reshape(1, H))
    acc0 = _sc_edge_pass(y0, srcp, dstp, zeros)
    y1, r1 = _mid_call(acc0, deg0, r0, Wl1, Wr1, b1.reshape(1, H))
    acc1 = _sc_edge_pass(y1, srcp, dstp, zeros)
    out = _fin_call(acc1, deg0, r1, lin_w, lin_b.reshape(1, 1))
    return out[:, 0]


# speedup vs baseline: 14.1306x; 1.0360x over previous
import functools

import jax
import jax.numpy as jnp
from jax import lax
from jax.experimental import pallas as pl
from jax.experimental.pallas import tpu as pltpu
from jax.experimental.pallas import tpu_sc as plsc

N = 10000
E = 320000
D = 128
H = 64
HH = H // 2

NC = 2
NS = 16
NW = NC * NS
CH = 125
GD = 80
G = 160
N_PAD = 10240
ROWS_PER_TILE = N_PAD // NS
YROWS_PER_TILE = N // NS
NB = 8
K = 4

_F32 = jnp.float32
_HIGH = jax.lax.Precision.HIGHEST


def _dot(a, b):
    return jnp.dot(a, b, preferred_element_type=_F32, precision=_HIGH)


_MESH = plsc.VectorSubcoreMesh(core_axis_name="c", subcore_axis_name="s")
_SC_PARAMS = pltpu.CompilerParams(use_tc_tiling_on_sc=False)


@functools.partial(
    pl.kernel,
    out_type=jax.ShapeDtypeStruct((NC, N_PAD, HH), _F32),
    mesh=_MESH,
    scratch_types=(
        pltpu.VMEM_SHARED((N_PAD, HH), _F32),
        pltpu.VMEM_SHARED((N, HH), _F32),
        pltpu.VMEM((G, CH), jnp.int32),
        pltpu.VMEM((G, CH), jnp.int32),
        pltpu.VMEM((NB, CH, HH), _F32),
        pltpu.SemaphoreType.DMA((NB,)),
        pltpu.SemaphoreType.DMA((NB,)),
    ),
    compiler_params=_SC_PARAMS)
def _sc_edge_pass(y_hbm, srcp, dstp, z_hbm, acc_out,
                  acc_sh, y_sh, src_v, dst_v, rows_v, gsem, ssem):
    c = lax.axis_index("c")
    s = lax.axis_index("s")
    row0 = s * ROWS_PER_TILE
    yrow0 = s * YROWS_PER_TILE

    pltpu.sync_copy(z_hbm.at[pl.ds(row0, ROWS_PER_TILE)],
                    acc_sh.at[pl.ds(row0, ROWS_PER_TILE)])
    pltpu.sync_copy(y_hbm.at[c, pl.ds(yrow0, YROWS_PER_TILE)],
                    y_sh.at[pl.ds(yrow0, YROWS_PER_TILE)])
    pltpu.sync_copy(srcp.at[s], src_v)
    pltpu.sync_copy(dstp.at[s], dst_v)
    plsc.subcore_barrier()

    for b in range(K):
        pltpu.async_copy(y_sh.at[src_v.at[b]], rows_v.at[b], gsem.at[b])

    def chunk(g, _):
        g2 = g + K
        s2 = lax.rem(g2, NB)

        @pl.when(g2 < G)
        def _prefetch():
            @pl.when(g >= K)
            def _drain_slot():
                pltpu.make_async_copy(
                    rows_v.at[s2], acc_sh.at[dst_v.at[g - K]],
                    ssem.at[s2]).wait()
            pltpu.async_copy(y_sh.at[src_v.at[g2]], rows_v.at[s2],
                             gsem.at[s2])

        b = lax.rem(g, NB)
        pltpu.make_async_copy(y_sh.at[src_v.at[g]], rows_v.at[b],
                              gsem.at[b]).wait()
        pltpu.async_copy(rows_v.at[b], acc_sh.at[dst_v.at[g]],
                         ssem.at[b], add=True)
        return 0

    lax.fori_loop(0, G, chunk, 0)
    for i in range(NB):
        pltpu.make_async_copy(rows_v.at[i], acc_sh.at[dst_v.at[G - NB + i]],
                              ssem.at[i]).wait()
    plsc.subcore_barrier()

    pltpu.sync_copy(acc_sh.at[pl.ds(row0, ROWS_PER_TILE)],
                    acc_out.at[c, pl.ds(row0, ROWS_PER_TILE)])


@functools.partial(
    pl.kernel,
    out_type=jax.ShapeDtypeStruct((NC, N_PAD, 16), _F32),
    mesh=_MESH,
    scratch_types=(
        pltpu.VMEM_SHARED((N_PAD, 16), _F32),
        pltpu.VMEM((GD, CH), jnp.int32),
        pltpu.VMEM((CH, 16), _F32),
        pltpu.SemaphoreType.DMA,
    ),
    compiler_params=_SC_PARAMS)
def _sc_deg_pass(dstp, z16_hbm, ones_hbm, deg_out, deg_sh, dst_v, ones_v,
                 dsem):
    c = lax.axis_index("c")
    s = lax.axis_index("s")
    w = c * NS + s
    row0 = s * ROWS_PER_TILE

    pltpu.sync_copy(z16_hbm.at[pl.ds(row0, ROWS_PER_TILE)],
                    deg_sh.at[pl.ds(row0, ROWS_PER_TILE)])
    pltpu.sync_copy(dstp.at[w], dst_v)
    pltpu.sync_copy(ones_hbm, ones_v)
    plsc.subcore_barrier()

    def fire(g, _):
        pltpu.async_copy(ones_v, deg_sh.at[dst_v.at[g]], dsem, add=True)
        return 0

    lax.fori_loop(0, GD, fire, 0)

    def drain(g, _):
        pltpu.make_async_copy(ones_v, deg_sh.at[dst_v.at[g]], dsem).wait()
        return 0

    lax.fori_loop(0, GD, drain, 0)
    plsc.subcore_barrier()

    pltpu.sync_copy(deg_sh.at[pl.ds(row0, ROWS_PER_TILE)],
                    deg_out.at[c, pl.ds(row0, ROWS_PER_TILE)])


def _pre_body(x_ref, wl_ref, wr_ref, b_ref, y_ref, r_ref):
    x = x_ref[...]
    y = _dot(x, wl_ref[...])
    y_ref[0] = y[:, :HH]
    y_ref[1] = y[:, HH:]
    r_ref[...] = _dot(x, wr_ref[...]) + b_ref[...]


def _mid_body(acc_ref, deg_ref, r_ref, wl_ref, wr_ref, b_ref, y_ref, r1_ref):
    acc = jnp.concatenate([acc_ref[0], acc_ref[1]], axis=1)
    deg = deg_ref[0, :, 0:1] + deg_ref[1, :, 0:1]
    mean = acc / jnp.maximum(deg, 1.0)
    h = jnp.maximum(mean + r_ref[...], 0.0)
    y = _dot(h, wl_ref[...])
    y_ref[0] = y[:, :HH]
    y_ref[1] = y[:, HH:]
    r1_ref[...] = _dot(h, wr_ref[...]) + b_ref[...]


def _fin_body(acc_ref, deg_ref, r_ref, w_ref, b_ref, o_ref):
    acc = jnp.concatenate([acc_ref[0], acc_ref[1]], axis=1)
    deg = deg_ref[0, :, 0:1] + deg_ref[1, :, 0:1]
    mean = acc / jnp.maximum(deg, 1.0)
    h = jnp.maximum(mean + r_ref[...], 0.0)
    o_ref[...] = _dot(h, w_ref[...]) + b_ref[...]


_pre_call = pl.pallas_call(
    _pre_body,
    out_shape=(jax.ShapeDtypeStruct((NC, N, HH), _F32),
               jax.ShapeDtypeStruct((N, H), _F32)))

_B = 2000

_mid_call = pl.pallas_call(
    _mid_body,
    grid=(N // _B,),
    in_specs=[
        pl.BlockSpec((NC, _B, HH), lambda i: (0, i, 0)),
        pl.BlockSpec((NC, _B, 16), lambda i: (0, i, 0)),
        pl.BlockSpec((_B, H), lambda i: (i, 0)),
        pl.BlockSpec((H, H), lambda i: (0, 0)),
        pl.BlockSpec((H, H), lambda i: (0, 0)),
        pl.BlockSpec((1, H), lambda i: (0, 0)),
    ],
    out_specs=(pl.BlockSpec((NC, _B, HH), lambda i: (0, i, 0)),
               pl.BlockSpec((_B, H), lambda i: (i, 0))),
    out_shape=(jax.ShapeDtypeStruct((NC, N, HH), _F32),
               jax.ShapeDtypeStruct((N, H), _F32)))

_fin_call = pl.pallas_call(
    _fin_body,
    grid=(N // _B,),
    in_specs=[
        pl.BlockSpec((NC, _B, HH), lambda i: (0, i, 0)),
        pl.BlockSpec((NC, _B, 16), lambda i: (0, i, 0)),
        pl.BlockSpec((_B, H), lambda i: (i, 0)),
        pl.BlockSpec((H, 1), lambda i: (0, 0)),
        pl.BlockSpec((1, 1), lambda i: (0, 0)),
    ],
    out_specs=pl.BlockSpec((_B, 1), lambda i: (i, 0)),
    out_shape=jax.ShapeDtypeStruct((N, 1), _F32))


def kernel(x, edge_index, Wl0, Wr0, b0, Wl1, Wr1, b1, lin_w, lin_b):
    srcp = edge_index[0].reshape(NS, G, CH)
    dstp = edge_index[1].reshape(NS, G, CH)
    dstp_deg = edge_index[1].reshape(NW, GD, CH)

    zeros = jnp.zeros((N_PAD, HH), _F32)
    zeros16 = jnp.zeros((N_PAD, 16), _F32)
    ones = jnp.ones((CH, 16), _F32)

    deg0 = _sc_deg_pass(dstp_deg, zeros16, ones)
    y0, r0 = _pre_call(x, Wl0, Wr0, b0.reshape(1, H))
    acc0 = _sc_edge_pass(y0, srcp, dstp, zeros)
    y1, r1 = _mid_call(acc0, deg0, r0, Wl1, Wr1, b1.reshape(1, H))
    acc1 = _sc_edge_pass(y1, srcp, dstp, zeros)
    out = _fin_call(acc1, deg0, r1, lin_w, lin_b.reshape(1, 1))
    return out[:, 0]
